# Initial kernel scaffold; baseline (speedup 1.0000x reference)
import jax
import jax.numpy as jnp
from jax.experimental import pallas as pl


def kernel(pt_fea, xyz_ind, g0, b0, w1, bl1, g1, be1, w2, bl2, g2, be2, w3, bl3, g3, be3, w4, bl4, wc, bc):
    raise NotImplementedError("write your pallas kernel here")


# speedup vs baseline: 3.2523x; 3.2523x over previous
import functools
import jax
import jax.numpy as jnp
from jax import lax
from jax.experimental import pallas as pl
from jax.experimental.pallas import tpu as pltpu

B, N, GX, GY = 2, 100000, 480, 360
FEA, RES = 9, 2
BN_PTS = B * N
VG = B * GX * GY
BLK = 4096
NBLK = (BN_PTS + BLK - 1) // BLK


def _stats_kernel(x_ref, stat_ref):
    i = pl.program_id(0)

    @pl.when(i == 0)
    def _():
        stat_ref[...] = jnp.zeros_like(stat_ref)

    x = x_ref[...]
    rows = jax.lax.broadcasted_iota(jnp.int32, (BLK, 1), 0) + i * BLK
    valid = rows < BN_PTS
    x = jnp.where(valid, x, 0.0)
    pos = x[:, :FEA]
    s = jnp.sum(pos, axis=0, keepdims=True)
    s2 = jnp.sum(pos * pos, axis=0, keepdims=True)
    stat_ref[0:1, :FEA] += s
    stat_ref[1:2, :FEA] += s2


def _layer_kernel(f_out, with_relu_affine, x_ref, w_ref, c_ref, a_ref,
                  t_ref, stat_ref):
    i = pl.program_id(0)

    @pl.when(i == 0)
    def _():
        stat_ref[...] = jnp.zeros_like(stat_ref)

    x = x_ref[...]
    if with_relu_affine:
        x = jnp.maximum(x * a_ref[0:1, :x.shape[1]] + a_ref[1:2, :x.shape[1]], 0.0)
    t = jnp.dot(x, w_ref[...], preferred_element_type=jnp.float32)
    t = t + c_ref[0:1, :t.shape[1]]
    t_ref[...] = t
    rows = jax.lax.broadcasted_iota(jnp.int32, (BLK, 1), 0) + i * BLK
    valid = rows < BN_PTS
    tm = jnp.where(valid, t, 0.0)
    stat_ref[0:1, :f_out] += jnp.sum(tm, axis=0, keepdims=True)
    stat_ref[1:2, :f_out] += jnp.sum(tm * tm, axis=0, keepdims=True)


def _final_kernel(x_ref, w_ref, c_ref, a_ref, t_ref):
    x = x_ref[...]
    x = jnp.maximum(x * a_ref[0:1, :x.shape[1]] + a_ref[1:2, :x.shape[1]], 0.0)
    t = jnp.dot(x, w_ref[...], preferred_element_type=jnp.float32)
    t_ref[...] = t + c_ref[0:1, :t.shape[1]]


def _run_stats(pt2d):
    return pl.pallas_call(
        _stats_kernel,
        grid=(NBLK,),
        in_specs=[pl.BlockSpec((BLK, FEA + RES), lambda i: (i, 0))],
        out_specs=pl.BlockSpec((8, 128), lambda i: (0, 0)),
        out_shape=jax.ShapeDtypeStruct((8, 128), jnp.float32),
    )(pt2d)


def _run_layer(x, w, c, a, f_out, with_relu_affine, f_in_blk):
    kfn = functools.partial(_layer_kernel, f_out, with_relu_affine)
    t, stat = pl.pallas_call(
        kfn,
        grid=(NBLK,),
        in_specs=[
            pl.BlockSpec((BLK, f_in_blk), lambda i: (i, 0)),
            pl.BlockSpec(w.shape, lambda i: (0, 0)),
            pl.BlockSpec(c.shape, lambda i: (0, 0)),
            pl.BlockSpec(a.shape, lambda i: (0, 0)),
        ],
        out_specs=[
            pl.BlockSpec((BLK, f_out), lambda i: (i, 0)),
            pl.BlockSpec((8, 128), lambda i: (0, 0)),
        ],
        out_shape=[
            jax.ShapeDtypeStruct((NBLK * BLK, f_out), jnp.float32),
            jax.ShapeDtypeStruct((8, 128), jnp.float32),
        ],
    )(x, w, c, a)
    return t, stat


def _run_final(x, w, c, a, f_out, f_in_blk):
    return pl.pallas_call(
        _final_kernel,
        grid=(NBLK,),
        in_specs=[
            pl.BlockSpec((BLK, f_in_blk), lambda i: (i, 0)),
            pl.BlockSpec(w.shape, lambda i: (0, 0)),
            pl.BlockSpec(c.shape, lambda i: (0, 0)),
            pl.BlockSpec(a.shape, lambda i: (0, 0)),
        ],
        out_specs=pl.BlockSpec((BLK, f_out), lambda i: (i, 0)),
        out_shape=jax.ShapeDtypeStruct((NBLK * BLK, f_out), jnp.float32),
    )(x, w, c, a)


def _bn_affine(stat, f, g, be):
    s = stat[0, :f]
    s2 = stat[1, :f]
    m = s / BN_PTS
    v = s2 / BN_PTS - m * m
    inv = g / jnp.sqrt(v + 1e-5)
    a0 = inv
    a1 = be - m * inv
    a = jnp.zeros((2, f), jnp.float32).at[0].set(a0).at[1].set(a1)
    return a


def kernel(pt_fea, xyz_ind, g0, b0, w1, bl1, g1, be1, w2, bl2, g2, be2,
           w3, bl3, g3, be3, w4, bl4, wc, bc):
    pt2d = pt_fea.reshape(BN_PTS, FEA + RES)

    stat0 = _run_stats(pt2d)
    s = stat0[0, :FEA]
    s2 = stat0[1, :FEA]
    m0 = s / BN_PTS
    v0 = s2 / BN_PTS - m0 * m0
    inv0 = g0 / jnp.sqrt(v0 + 1e-5)
    w1f = jnp.zeros((FEA + RES, 32), jnp.float32).at[:FEA].set(inv0[:, None] * w1)
    c1f = ((b0 - m0 * inv0) @ w1 + bl1)[None, :]
    dummy_a = jnp.zeros((2, FEA + RES), jnp.float32)
    t1, stat1 = _run_layer(pt2d, w1f, c1f, dummy_a, 32, False, FEA + RES)

    a1 = _bn_affine(stat1, 32, g1, be1)
    t2, stat2 = _run_layer(t1, w2, bl2[None, :], a1, 64, True, 32)

    a2 = _bn_affine(stat2, 64, g2, be2)
    t3, stat3 = _run_layer(t2, w3, bl3[None, :], a2, 128, True, 64)

    a3 = _bn_affine(stat3, 128, g3, be3)
    proc = _run_final(t3, w4, bl4[None, :], a3, 256, 128)
    proc = proc[:BN_PTS]

    cat = pt2d
    res = cat[:, FEA:]
    bidx = jnp.repeat(jnp.arange(B, dtype=jnp.int32), N)
    xy = xyz_ind.reshape(BN_PTS, 2).astype(jnp.int32)
    vox = bidx * (GX * GY) + xy[:, 0] * GY + xy[:, 1]

    neg = jnp.float32(-jnp.inf)
    pooled = jnp.full((VG, 256), neg).at[vox].max(proc)
    pooledr = jnp.full((VG, RES), neg).at[vox].max(res)
    occ = pooled[:, 0] > neg
    pm = jnp.where(occ[:, None], pooled, 0.0)
    comp = jax.nn.relu(pm @ wc + bc) * occ[:, None]
    resd = jnp.where(occ[:, None], pooledr, 0.0)

    out = comp.reshape(B, GX, GY, 32).transpose(0, 3, 1, 2)
    out = jax.lax.reduce_window(out, -jnp.inf, jax.lax.max,
                                (1, 1, 3, 3), (1, 1, 1, 1), 'SAME')
    resd = resd.reshape(B, GX, GY, RES).transpose(0, 3, 1, 2)
    return jnp.concatenate([out, resd], axis=1)
